# baseline (speedup 1.0000x reference)
import functools

import jax
import jax.numpy as jnp
from jax import lax
from jax.experimental import pallas as pl
from jax.experimental.pallas import tpu as pltpu
from jax.experimental.pallas import tpu_sc as plsc


def _gather_call(table, idx_flat, *, num_workers, chunk, num_chunks, dim):
    per_w = chunk * num_chunks
    total = per_w * num_workers
    mesh = plsc.VectorSubcoreMesh(core_axis_name="c", subcore_axis_name="s")

    @functools.partial(
        pl.kernel,
        out_type=jax.ShapeDtypeStruct((total, dim), jnp.float32),
        mesh=mesh,
        scratch_types=[
            pltpu.VMEM((per_w,), jnp.int32),
            pltpu.VMEM((2, chunk, dim), jnp.float32),
            pltpu.SemaphoreType.DMA,
            pltpu.SemaphoreType.DMA,
        ],
        compiler_params=pltpu.CompilerParams(use_tc_tiling_on_sc=False),
    )
    def gather_kernel(table_hbm, idx_hbm, out_hbm, idx_v, rows_v, sem0, sem1):
        wid = lax.axis_index("s") * 2 + lax.axis_index("c")
        base = wid * per_w
        pltpu.sync_copy(idx_hbm.at[pl.ds(base, per_w)], idx_v)

        sems = (sem0, sem1)

        def start(j, buf):
            idx_s = idx_v.at[pl.ds(j * chunk, chunk)]
            pltpu.async_copy(table_hbm.at[idx_s], rows_v.at[buf], sems[buf])

        def wait(j, buf):
            idx_s = idx_v.at[pl.ds(j * chunk, chunk)]
            pltpu.make_async_copy(
                table_hbm.at[idx_s], rows_v.at[buf], sems[buf]
            ).wait()

        start(0, 0)

        @pl.loop(0, num_chunks, step=2)
        def _(j):
            for b in range(2):
                jj = j + b

                @pl.when(jj + 1 < num_chunks)
                def _():
                    start(jj + 1, (b + 1) % 2)

                wait(jj, b)
                pltpu.sync_copy(
                    rows_v.at[b], out_hbm.at[pl.ds(base + jj * chunk, chunk)]
                )

    return gather_kernel(table, idx_flat)


def kernel(input, index):
    n_rows, dim = input.shape
    b0, b1 = index.shape
    total = b0 * b1

    num_workers = 32
    chunk = 512
    per_w = total // num_workers
    num_chunks = per_w // chunk
    assert per_w * num_workers == total and chunk * num_chunks == per_w

    idx_flat = index.reshape(total)
    out = _gather_call(
        input,
        idx_flat,
        num_workers=num_workers,
        chunk=chunk,
        num_chunks=num_chunks,
        dim=dim,
    )
    return out.reshape(b0, b1, dim)
